# Initial kernel scaffold; baseline (speedup 1.0000x reference)
import jax
import jax.numpy as jnp
from jax.experimental import pallas as pl


def kernel(input_ids, word_emb, type_emb, pos_emb, ln_gamma, ln_beta, W_map, b_map):
    raise NotImplementedError("write your pallas kernel here")


# speedup vs baseline: 1.5837x; 1.5837x over previous
import functools

import jax
import jax.numpy as jnp
from jax import lax
from jax.experimental import pallas as pl
from jax.experimental.pallas import tpu as pltpu
from jax.experimental.pallas import tpu_sc as plsc

_EPS = 1e-12
_NC, _NS = 2, 16
_NW = _NC * _NS
_IDX_CHUNK = 128
_TN = 512


def _sc_gather(word_emb, ids2d):
    n_rows, chunk = ids2d.shape
    n = n_rows * chunk
    d = word_emb.shape[1]
    per_w = n // _NW
    chunks_per_w = per_w // chunk
    mesh = plsc.VectorSubcoreMesh(core_axis_name="c", subcore_axis_name="s")

    @functools.partial(
        pl.kernel,
        mesh=mesh,
        out_type=jax.ShapeDtypeStruct((n, d), word_emb.dtype),
        scratch_types=[
            pltpu.VMEM((chunks_per_w, chunk), jnp.int32),
            pltpu.VMEM((per_w, d), word_emb.dtype),
            pltpu.SemaphoreType.DMA,
        ],
    )
    def gather_kernel(table_hbm, idx_hbm, out_hbm, idx_v, rows_v, sem):
        wid = lax.axis_index("s") * _NC + lax.axis_index("c")
        pltpu.sync_copy(idx_hbm.at[pl.ds(wid * chunks_per_w, chunks_per_w)], idx_v)
        copies = [
            pltpu.async_copy(
                table_hbm.at[idx_v.at[j]],
                rows_v.at[pl.ds(j * chunk, chunk)],
                sem,
            )
            for j in range(chunks_per_w)
        ]
        for c in copies:
            c.wait()
        pltpu.sync_copy(rows_v, out_hbm.at[pl.ds(wid * per_w, per_w)])

    return gather_kernel(word_emb, ids2d)


def _dense_body(g_ref, pos_ref, type_ref, gam_ref, bet_ref, w_ref, b_ref, o_ref):
    x = g_ref[...] + pos_ref[...] + type_ref[0:1, :]
    mu = jnp.mean(x, axis=1, keepdims=True)
    xc = x - mu
    var = jnp.mean(xc * xc, axis=1, keepdims=True)
    nrm = xc * lax.rsqrt(var + _EPS) * gam_ref[...] + bet_ref[...]
    o_ref[...] = (
        jnp.dot(nrm, w_ref[...], preferred_element_type=jnp.float32) + b_ref[...]
    )


def _tc_dense(gathered, pos_emb, type_emb, ln_gamma, ln_beta, w_map, b_map, seq_len):
    n, d = gathered.shape
    h = w_map.shape[1]
    blocks_per_seq = seq_len // _TN
    grid = (n // _TN,)
    return pl.pallas_call(
        _dense_body,
        grid=grid,
        in_specs=[
            pl.BlockSpec((_TN, d), lambda i: (i, 0)),
            pl.BlockSpec((_TN, d), lambda i: (i % blocks_per_seq, 0)),
            pl.BlockSpec(type_emb.shape, lambda i: (0, 0)),
            pl.BlockSpec((1, d), lambda i: (0, 0)),
            pl.BlockSpec((1, d), lambda i: (0, 0)),
            pl.BlockSpec((d, h), lambda i: (0, 0)),
            pl.BlockSpec((1, h), lambda i: (0, 0)),
        ],
        out_specs=pl.BlockSpec((_TN, h), lambda i: (i, 0)),
        out_shape=jax.ShapeDtypeStruct((n, h), jnp.float32),
        compiler_params=pltpu.CompilerParams(
            dimension_semantics=("parallel",),
        ),
    )(
        gathered,
        pos_emb,
        type_emb,
        ln_gamma.reshape(1, d),
        ln_beta.reshape(1, d),
        w_map,
        b_map.reshape(1, h),
    )


def kernel(input_ids, word_emb, type_emb, pos_emb, ln_gamma, ln_beta, W_map, b_map):
    b, s = input_ids.shape
    n = b * s
    h = W_map.shape[1]
    ids2d = input_ids.reshape(n // _IDX_CHUNK, _IDX_CHUNK).astype(jnp.int32)
    gathered = _sc_gather(word_emb, ids2d)
    out = _tc_dense(
        gathered, pos_emb, type_emb, ln_gamma, ln_beta, W_map, b_map, s
    )
    return out.reshape(b, s, h)


# speedup vs baseline: 1.6232x; 1.0250x over previous
import functools

import jax
import jax.numpy as jnp
from jax import lax
from jax.experimental import pallas as pl
from jax.experimental.pallas import tpu as pltpu
from jax.experimental.pallas import tpu_sc as plsc

_EPS = 1e-12
_NC, _NS = 2, 16
_NW = _NC * _NS
_IDX_CHUNK = 128
_TN = 512


def _sc_gather(word_emb, ids2d):
    n_rows, chunk = ids2d.shape
    n = n_rows * chunk
    d = word_emb.shape[1]
    per_w = n // _NW
    chunks_per_w = per_w // chunk
    mesh = plsc.VectorSubcoreMesh(core_axis_name="c", subcore_axis_name="s")

    @functools.partial(
        pl.kernel,
        mesh=mesh,
        out_type=jax.ShapeDtypeStruct((n, d), word_emb.dtype),
        scratch_types=[
            pltpu.VMEM((chunks_per_w, chunk), jnp.int32),
            pltpu.VMEM((per_w, d), word_emb.dtype),
            pltpu.SemaphoreType.DMA,
        ],
    )
    def gather_kernel(table_hbm, idx_hbm, out_hbm, idx_v, rows_v, sem):
        wid = lax.axis_index("s") * _NC + lax.axis_index("c")
        pltpu.sync_copy(idx_hbm.at[pl.ds(wid * chunks_per_w, chunks_per_w)], idx_v)
        copies = [
            pltpu.async_copy(
                table_hbm.at[idx_v.at[j]],
                rows_v.at[pl.ds(j * chunk, chunk)],
                sem,
            )
            for j in range(chunks_per_w)
        ]
        for c in copies:
            c.wait()
        pltpu.sync_copy(rows_v, out_hbm.at[pl.ds(wid * per_w, per_w)])

    return gather_kernel(word_emb, ids2d)


def _dense_body(g_ref, pos_ref, type_ref, gam_ref, bet_ref, w_ref, b_ref, o_ref):
    x = g_ref[...] + pos_ref[...] + type_ref[0:1, :]
    mu = jnp.mean(x, axis=1, keepdims=True)
    xc = x - mu
    var = jnp.mean(xc * xc, axis=1, keepdims=True)
    nrm = xc * lax.rsqrt(var + _EPS) * gam_ref[...] + bet_ref[...]
    o_ref[...] = (
        jnp.dot(nrm, w_ref[...], preferred_element_type=jnp.float32) + b_ref[...]
    )


def _tc_dense(gathered, pos_emb, type_emb, ln_gamma, ln_beta, w_map, b_map, seq_len):
    n, d = gathered.shape
    h = w_map.shape[1]
    blocks_per_seq = seq_len // _TN
    n_batch = n // seq_len
    grid = (blocks_per_seq, n_batch)
    return pl.pallas_call(
        _dense_body,
        grid=grid,
        in_specs=[
            pl.BlockSpec((_TN, d), lambda p, b: (b * blocks_per_seq + p, 0)),
            pl.BlockSpec((_TN, d), lambda p, b: (p, 0)),
            pl.BlockSpec(type_emb.shape, lambda p, b: (0, 0)),
            pl.BlockSpec((1, d), lambda p, b: (0, 0)),
            pl.BlockSpec((1, d), lambda p, b: (0, 0)),
            pl.BlockSpec((d, h), lambda p, b: (0, 0)),
            pl.BlockSpec((1, h), lambda p, b: (0, 0)),
        ],
        out_specs=pl.BlockSpec((_TN, h), lambda p, b: (b * blocks_per_seq + p, 0)),
        out_shape=jax.ShapeDtypeStruct((n, h), jnp.float32),
        compiler_params=pltpu.CompilerParams(
            dimension_semantics=("parallel", "parallel"),
        ),
    )(
        gathered,
        pos_emb,
        type_emb,
        ln_gamma.reshape(1, d),
        ln_beta.reshape(1, d),
        w_map,
        b_map.reshape(1, h),
    )


def kernel(input_ids, word_emb, type_emb, pos_emb, ln_gamma, ln_beta, W_map, b_map):
    b, s = input_ids.shape
    n = b * s
    h = W_map.shape[1]
    ids2d = input_ids.reshape(n // _IDX_CHUNK, _IDX_CHUNK).astype(jnp.int32)
    gathered = _sc_gather(word_emb, ids2d)
    out = _tc_dense(
        gathered, pos_emb, type_emb, ln_gamma, ln_beta, W_map, b_map, s
    )
    return out.reshape(b, s, h)
